# Initial kernel scaffold; baseline (speedup 1.0000x reference)
import jax
import jax.numpy as jnp
from jax.experimental import pallas as pl


def kernel(x, edge_index, W1, a_src1, a_dst1, b1, W2, a_src2, a_dst2, b2):
    raise NotImplementedError("write your pallas kernel here")


# speedup vs baseline: 34.4011x; 34.4011x over previous
import functools

import jax
import jax.numpy as jnp
from jax import lax
from jax.experimental import pallas as pl
from jax.experimental.pallas import tpu as pltpu
from jax.experimental.pallas import tpu_sc as plsc

_N = 10000
_E = 320000
_D = 128
_H1 = 8
_C1 = 8
_NC = 64

_NCORES = 2
_NSUB = 16
_CHUNK = 80
_BN = 1000


def _make_edge_pass(row_src, row_dst, row_acc, per_head):
    e_per_sc = _E // _NCORES
    e_per_tile = e_per_sc // _NSUB
    n_chunks = e_per_tile // _CHUNK
    rows_u = 624
    tail0 = rows_u * _NSUB
    tail_rows = _N - tail0
    zrows = 156

    mesh = plsc.VectorSubcoreMesh(core_axis_name="c", subcore_axis_name="s",
                                  num_cores=_NCORES, num_subcores=_NSUB)

    @functools.partial(
        pl.kernel,
        out_type=jax.ShapeDtypeStruct((_NCORES, _N, row_acc), jnp.float32),
        mesh=mesh,
        compiler_params=pltpu.CompilerParams(use_tc_tiling_on_sc=False),
        scratch_types=[
            pltpu.VMEM((_CHUNK,), jnp.int32),
            pltpu.VMEM((_CHUNK,), jnp.int32),
            pltpu.VMEM((_CHUNK, row_src), jnp.float32),
            pltpu.VMEM((_CHUNK, row_dst), jnp.float32),
            pltpu.VMEM((_CHUNK, row_acc), jnp.float32),
            pltpu.VMEM((zrows, row_acc), jnp.float32),
            pltpu.VMEM_SHARED((_N, row_acc), jnp.float32),
        ],
    )
    def edge_kernel(src_h, dst_h, bysrc_h, bydst_h, out_h,
                    src_i, dst_i, rsrc, rdst, msg, zbuf, acc):
        cid = lax.axis_index("c")
        sid = lax.axis_index("s")
        row0 = sid * rows_u

        zero16 = jnp.zeros((16,), jnp.float32)

        def zero_row(i, _):
            for j in range(row_acc // 16):
                zbuf[i, pl.ds(16 * j, 16)] = zero16
            return 0

        lax.fori_loop(0, zrows, zero_row, 0)
        for k in range(rows_u // zrows):
            pltpu.sync_copy(zbuf, acc.at[pl.ds(row0 + k * zrows, zrows)])

        @pl.when(sid == _NSUB - 1)
        def _():
            pltpu.sync_copy(zbuf.at[pl.ds(0, tail_rows)],
                            acc.at[pl.ds(tail0, tail_rows)])

        plsc.subcore_barrier()

        tile_base = cid * e_per_sc + sid * e_per_tile

        def edge_body(ei, _):
            if per_head:
                for j in range(4):
                    sl = pl.ds(16 * j, 16)
                    a = rsrc[ei, pl.ds(64 + 16 * j, 16)] + rdst[ei, sl]
                    w = jnp.exp(jnp.maximum(a, 0.2 * a))
                    msg[ei, sl] = rsrc[ei, sl] * w
                    msg[ei, pl.ds(64 + 16 * j, 16)] = w
            else:
                a = rsrc[ei, pl.ds(64, 16)] + rdst[ei, pl.ds(0, 16)]
                w = jnp.exp(jnp.maximum(a, 0.2 * a))
                for j in range(4):
                    sl = pl.ds(16 * j, 16)
                    msg[ei, sl] = rsrc[ei, sl] * w
                msg[ei, pl.ds(64, 16)] = w
            return 0

        def chunk_body(i, _):
            base = tile_base + i * _CHUNK
            pltpu.sync_copy(src_h.at[pl.ds(base, _CHUNK)], src_i)
            pltpu.sync_copy(dst_h.at[pl.ds(base, _CHUNK)], dst_i)
            pltpu.sync_copy(bysrc_h.at[src_i], rsrc)
            pltpu.sync_copy(bydst_h.at[dst_i], rdst)
            lax.fori_loop(0, _CHUNK, edge_body, 0)
            pltpu.sync_copy(msg, acc.at[dst_i], add=True)
            return 0

        lax.fori_loop(0, n_chunks, chunk_body, 0)
        plsc.subcore_barrier()
        pltpu.sync_copy(acc.at[pl.ds(row0, rows_u)],
                        out_h.at[cid, pl.ds(row0, rows_u)])

        @pl.when(sid == _NSUB - 1)
        def _():
            pltpu.sync_copy(acc.at[pl.ds(tail0, tail_rows)],
                            out_h.at[cid, pl.ds(tail0, tail_rows)])

    return edge_kernel


def _prep1(x, A1, B1):
    def body(x_ref, a_ref, b_ref, o1_ref, o2_ref):
        xv = x_ref[...]
        o1_ref[...] = jnp.dot(xv, a_ref[...], preferred_element_type=jnp.float32)
        o2_ref[...] = jnp.dot(xv, b_ref[...], preferred_element_type=jnp.float32)

    return pl.pallas_call(
        body,
        grid=(_N // _BN,),
        in_specs=[
            pl.BlockSpec((_BN, _D), lambda i: (i, 0)),
            pl.BlockSpec((_D, 128), lambda i: (0, 0)),
            pl.BlockSpec((_D, 64), lambda i: (0, 0)),
        ],
        out_specs=[
            pl.BlockSpec((_BN, 128), lambda i: (i, 0)),
            pl.BlockSpec((_BN, 64), lambda i: (i, 0)),
        ],
        out_shape=[
            jax.ShapeDtypeStruct((_N, 128), jnp.float32),
            jax.ShapeDtypeStruct((_N, 64), jnp.float32),
        ],
    )(x, A1, B1)


def _finalize1_prep2(acc1, bysrc1, bydst1, b1, A2, B2):
    def body(acc_ref, bs_ref, bd_ref, b1_ref, a2_ref, b2_ref, o1_ref, o2_ref):
        num = acc_ref[0, :, 0:64] + acc_ref[1, :, 0:64]
        den = acc_ref[0, :, 64:128] + acc_ref[1, :, 64:128]
        h = bs_ref[:, 0:64]
        a0 = bs_ref[:, 64:128] + bd_ref[...]
        ws = jnp.exp(jnp.maximum(a0, 0.2 * a0))
        o = (num + ws * h) / (den + ws + 1e-16) + b1_ref[...]
        h1 = jnp.where(o > 0, o, jnp.exp(o) - 1.0)
        o1_ref[...] = jnp.dot(h1, a2_ref[...], preferred_element_type=jnp.float32)
        o2_ref[...] = jnp.dot(h1, b2_ref[...], preferred_element_type=jnp.float32)

    return pl.pallas_call(
        body,
        grid=(_N // _BN,),
        in_specs=[
            pl.BlockSpec((_NCORES, _BN, 128), lambda i: (0, i, 0)),
            pl.BlockSpec((_BN, 128), lambda i: (i, 0)),
            pl.BlockSpec((_BN, 64), lambda i: (i, 0)),
            pl.BlockSpec((1, 64), lambda i: (0, 0)),
            pl.BlockSpec((64, 80), lambda i: (0, 0)),
            pl.BlockSpec((64, 16), lambda i: (0, 0)),
        ],
        out_specs=[
            pl.BlockSpec((_BN, 80), lambda i: (i, 0)),
            pl.BlockSpec((_BN, 16), lambda i: (i, 0)),
        ],
        out_shape=[
            jax.ShapeDtypeStruct((_N, 80), jnp.float32),
            jax.ShapeDtypeStruct((_N, 16), jnp.float32),
        ],
    )(acc1, bysrc1, bydst1, b1, A2, B2)


def _finalize2(acc2, bysrc2, bydst2, b2):
    def body(acc_ref, bs_ref, bd_ref, b2_ref, o_ref):
        num = acc_ref[0, :, 0:64] + acc_ref[1, :, 0:64]
        den = acc_ref[0, :, 64:65] + acc_ref[1, :, 64:65]
        h = bs_ref[:, 0:64]
        a0 = bs_ref[:, 64:65] + bd_ref[:, 0:1]
        ws = jnp.exp(jnp.maximum(a0, 0.2 * a0))
        o = (num + ws * h) / (den + ws + 1e-16) + b2_ref[...]
        m = jnp.max(o, axis=1, keepdims=True)
        z = o - m
        o_ref[...] = z - jnp.log(jnp.sum(jnp.exp(z), axis=1, keepdims=True))

    return pl.pallas_call(
        body,
        grid=(_N // _BN,),
        in_specs=[
            pl.BlockSpec((_NCORES, _BN, 80), lambda i: (0, i, 0)),
            pl.BlockSpec((_BN, 80), lambda i: (i, 0)),
            pl.BlockSpec((_BN, 16), lambda i: (i, 0)),
            pl.BlockSpec((1, 64), lambda i: (0, 0)),
        ],
        out_specs=pl.BlockSpec((_BN, 64), lambda i: (i, 0)),
        out_shape=jax.ShapeDtypeStruct((_N, 64), jnp.float32),
    )(acc2, bysrc2, bydst2, b2)


def _expand_mat(a):
    hh, cc = a.shape
    t = a[:, :, None, None] * jnp.eye(hh, dtype=a.dtype)[:, None, :, None]
    t = jnp.broadcast_to(t, (hh, cc, hh, cc))
    return t.reshape(hh * cc, hh * cc)


_edge_pass_cache = {}


def _edge_pass(row_src, row_dst, row_acc, per_head):
    key = (row_src, row_dst, row_acc, per_head)
    if key not in _edge_pass_cache:
        _edge_pass_cache[key] = _make_edge_pass(row_src, row_dst, row_acc,
                                                per_head)
    return _edge_pass_cache[key]


def kernel(x, edge_index, W1, a_src1, a_dst1, b1, W2, a_src2, a_dst2, b2):
    src = edge_index[0]
    dst = edge_index[1]

    Msrc1 = _expand_mat(a_src1)
    Mdst1 = _expand_mat(a_dst1)
    A1 = W1 @ jnp.concatenate([jnp.eye(64, dtype=W1.dtype), Msrc1], axis=1)
    B1 = W1 @ Mdst1
    s2 = W2 @ a_src2.T
    d2 = W2 @ a_dst2.T
    A2 = jnp.concatenate([W2, jnp.broadcast_to(s2, (64, 16))], axis=1)
    B2 = jnp.broadcast_to(d2, (64, 16))

    bysrc1, bydst1 = _prep1(x, A1, B1)
    acc1 = _edge_pass(128, 64, 128, True)(src, dst, bysrc1, bydst1)
    bysrc2, bydst2 = _finalize1_prep2(acc1, bysrc1, bydst1,
                                      b1.reshape(1, 64), A2, B2)
    acc2 = _edge_pass(80, 16, 80, False)(src, dst, bysrc2, bydst2)
    return _finalize2(acc2, bysrc2, bydst2, b2.reshape(1, 64))


# speedup vs baseline: 60.4112x; 1.7561x over previous
import functools

import jax
import jax.numpy as jnp
from jax import lax
from jax.experimental import pallas as pl
from jax.experimental.pallas import tpu as pltpu
from jax.experimental.pallas import tpu_sc as plsc

_N = 10000
_E = 320000
_D = 128

_NCORES = 2
_NSUB = 16
_CHUNK = 80
_BN = 1000

_RS = 80
_RD = 16
_RA = 80


def _make_edge_pass(per_head):
    e_per_sc = _E // _NCORES
    e_per_tile = e_per_sc // _NSUB
    n_chunks = e_per_tile // _CHUNK
    rows_u = 624
    tail0 = rows_u * _NSUB
    tail_rows = _N - tail0
    zrows = 48

    mesh = plsc.VectorSubcoreMesh(core_axis_name="c", subcore_axis_name="s",
                                  num_cores=_NCORES, num_subcores=_NSUB)

    @functools.partial(
        pl.kernel,
        out_type=jax.ShapeDtypeStruct((_NCORES, _N, _RA), jnp.float32),
        mesh=mesh,
        compiler_params=pltpu.CompilerParams(use_tc_tiling_on_sc=False,
                                             needs_layout_passes=False),
        scratch_types=[
            pltpu.VMEM((n_chunks, _CHUNK), jnp.int32),
            pltpu.VMEM((n_chunks, _CHUNK), jnp.int32),
            pltpu.VMEM((_CHUNK, _RS), jnp.float32),
            pltpu.VMEM((_CHUNK, _RS), jnp.float32),
            pltpu.VMEM((_CHUNK, _RD), jnp.float32),
            pltpu.VMEM((_CHUNK, _RD), jnp.float32),
            pltpu.VMEM((_CHUNK, _RA), jnp.float32),
            pltpu.VMEM((_CHUNK, _RA), jnp.float32),
            pltpu.VMEM((zrows, _RA), jnp.float32),
            pltpu.VMEM((4, 16), jnp.int32),
            pltpu.VMEM_SHARED((_N, _RA), jnp.float32),
            pltpu.SemaphoreType.DMA,
            pltpu.SemaphoreType.DMA,
            pltpu.SemaphoreType.DMA,
            pltpu.SemaphoreType.DMA,
        ],
    )
    def edge_kernel(src_h, dst_h, bysrc_h, bydst_h, lanes_h, out_h,
                    src_i, dst_i, rsrc0, rsrc1, rdst0, rdst1, msg0, msg1,
                    zbuf, lanes_v, acc, gsem0, gsem1, ssem0, ssem1):
        cid = lax.axis_index("c")
        sid = lax.axis_index("s")
        tid = cid * _NSUB + sid
        row0 = sid * rows_u
        rsrc = (rsrc0, rsrc1)
        rdst = (rdst0, rdst1)
        msg = (msg0, msg1)
        gsem = (gsem0, gsem1)
        ssem = (ssem0, ssem1)

        idx_copies = [
            pltpu.async_copy(src_h.at[tid], src_i, gsem0),
            pltpu.async_copy(dst_h.at[tid], dst_i, gsem1),
        ]
        pltpu.sync_copy(lanes_h, lanes_v)

        zero16 = jnp.zeros((16,), jnp.float32)

        def zero_row(i, _):
            for j in range(_RA // 16):
                zbuf[i, pl.ds(16 * j, 16)] = zero16
            return 0

        lax.fori_loop(0, zrows, zero_row, 0)
        for k in range(rows_u // zrows):
            pltpu.sync_copy(zbuf, acc.at[pl.ds(row0 + k * zrows, zrows)])

        @pl.when(sid == _NSUB - 1)
        def _():
            pltpu.sync_copy(zbuf.at[pl.ds(0, tail_rows)],
                            acc.at[pl.ds(tail0, tail_rows)])

        for c in idx_copies:
            c.wait()
        plsc.subcore_barrier()

        def issue(c, b):
            pltpu.async_copy(bysrc_h.at[src_i.at[c]], rsrc[b], gsem[b])
            pltpu.async_copy(bydst_h.at[dst_i.at[c]], rdst[b], gsem[b])

        def wait_gathers(b):
            pltpu.make_async_copy(bysrc_h.at[src_i.at[0]], rsrc[b],
                                  gsem[b]).wait()
            pltpu.make_async_copy(bydst_h.at[dst_i.at[0]], rdst[b],
                                  gsem[b]).wait()

        def wait_scatter(b):
            pltpu.make_async_copy(msg[b], acc.at[dst_i.at[0]],
                                  ssem[b]).wait()

        def edge_body(b):
            def body(ei, _):
                a = rsrc[b][ei, pl.ds(64, 16)] + rdst[b][ei, pl.ds(0, 16)]
                w = jnp.exp(jnp.maximum(a, 0.2 * a))
                msg[b][ei, pl.ds(64, 16)] = w
                row_idx = jnp.full((16,), ei, jnp.int32)
                for j in range(4):
                    sl = pl.ds(16 * j, 16)
                    if per_head:
                        wj = plsc.load_gather(msg[b], [row_idx, lanes_v[j, :]])
                    else:
                        wj = w
                    msg[b][ei, sl] = rsrc[b][ei, sl] * wj
                return 0
            lax.fori_loop(0, _CHUNK, body, 0)

        issue(0, 0)
        issue(1, 1)
        wait_gathers(0)
        edge_body(0)
        pltpu.async_copy(msg0, acc.at[dst_i.at[0]], ssem0, add=True)

        def pair_body(k, _):
            c = 2 * k + 1

            @pl.when(c + 1 < n_chunks)
            def _():
                issue(c + 1, 0)
            wait_gathers(1)

            @pl.when(k > 0)
            def _():
                wait_scatter(1)
            edge_body(1)
            pltpu.async_copy(msg1, acc.at[dst_i.at[c]], ssem1, add=True)

            @pl.when(c + 2 < n_chunks)
            def _():
                issue(c + 2, 1)

            @pl.when(c + 1 < n_chunks)
            def _():
                wait_gathers(0)
                wait_scatter(0)
                edge_body(0)
                pltpu.async_copy(msg0, acc.at[dst_i.at[c + 1]], ssem0,
                                 add=True)
            return 0

        lax.fori_loop(0, (n_chunks - 1) // 2, pair_body, 0)
        wait_scatter(0)
        wait_scatter(1)
        plsc.subcore_barrier()
        pltpu.sync_copy(acc.at[pl.ds(row0, rows_u)],
                        out_h.at[cid, pl.ds(row0, rows_u)])

        @pl.when(sid == _NSUB - 1)
        def _():
            pltpu.sync_copy(acc.at[pl.ds(tail0, tail_rows)],
                            out_h.at[cid, pl.ds(tail0, tail_rows)])

    return edge_kernel


def _prep1(x, A1, B1):
    def body(x_ref, a_ref, b_ref, o1_ref, o2_ref):
        xv = x_ref[...]
        o1_ref[...] = jnp.dot(xv, a_ref[...], preferred_element_type=jnp.float32)
        o2_ref[...] = jnp.dot(xv, b_ref[...], preferred_element_type=jnp.float32)

    return pl.pallas_call(
        body,
        grid=(_N // _BN,),
        in_specs=[
            pl.BlockSpec((_BN, _D), lambda i: (i, 0)),
            pl.BlockSpec((_D, _RS), lambda i: (0, 0)),
            pl.BlockSpec((_D, _RD), lambda i: (0, 0)),
        ],
        out_specs=[
            pl.BlockSpec((_BN, _RS), lambda i: (i, 0)),
            pl.BlockSpec((_BN, _RD), lambda i: (i, 0)),
        ],
        out_shape=[
            jax.ShapeDtypeStruct((_N, _RS), jnp.float32),
            jax.ShapeDtypeStruct((_N, _RD), jnp.float32),
        ],
    )(x, A1, B1)


def _finalize1_prep2(acc1, bysrc1, bydst1, b1, E16, A2, B2):
    def body(acc_ref, bs_ref, bd_ref, b1_ref, e_ref, a2_ref, b2_ref,
             o1_ref, o2_ref):
        num = acc_ref[0, :, 0:64] + acc_ref[1, :, 0:64]
        den16 = acc_ref[0, :, 64:80] + acc_ref[1, :, 64:80]
        ev = e_ref[...]
        den = jnp.dot(den16, ev, preferred_element_type=jnp.float32)
        h = bs_ref[:, 0:64]
        a0 = bs_ref[:, 64:80] + bd_ref[...]
        ws16 = jnp.exp(jnp.maximum(a0, 0.2 * a0))
        ws = jnp.dot(ws16, ev, preferred_element_type=jnp.float32)
        o = (num + ws * h) / (den + ws + 1e-16) + b1_ref[...]
        h1 = jnp.where(o > 0, o, jnp.exp(o) - 1.0)
        o1_ref[...] = jnp.dot(h1, a2_ref[...], preferred_element_type=jnp.float32)
        o2_ref[...] = jnp.dot(h1, b2_ref[...], preferred_element_type=jnp.float32)

    return pl.pallas_call(
        body,
        grid=(_N // _BN,),
        in_specs=[
            pl.BlockSpec((_NCORES, _BN, _RA), lambda i: (0, i, 0)),
            pl.BlockSpec((_BN, _RS), lambda i: (i, 0)),
            pl.BlockSpec((_BN, _RD), lambda i: (i, 0)),
            pl.BlockSpec((1, 64), lambda i: (0, 0)),
            pl.BlockSpec((16, 64), lambda i: (0, 0)),
            pl.BlockSpec((64, _RS), lambda i: (0, 0)),
            pl.BlockSpec((64, _RD), lambda i: (0, 0)),
        ],
        out_specs=[
            pl.BlockSpec((_BN, _RS), lambda i: (i, 0)),
            pl.BlockSpec((_BN, _RD), lambda i: (i, 0)),
        ],
        out_shape=[
            jax.ShapeDtypeStruct((_N, _RS), jnp.float32),
            jax.ShapeDtypeStruct((_N, _RD), jnp.float32),
        ],
    )(acc1, bysrc1, bydst1, b1, E16, A2, B2)


def _finalize2(acc2, bysrc2, bydst2, b2):
    def body(acc_ref, bs_ref, bd_ref, b2_ref, o_ref):
        num = acc_ref[0, :, 0:64] + acc_ref[1, :, 0:64]
        den = acc_ref[0, :, 64:65] + acc_ref[1, :, 64:65]
        h = bs_ref[:, 0:64]
        a0 = bs_ref[:, 64:65] + bd_ref[:, 0:1]
        ws = jnp.exp(jnp.maximum(a0, 0.2 * a0))
        o = (num + ws * h) / (den + ws + 1e-16) + b2_ref[...]
        m = jnp.max(o, axis=1, keepdims=True)
        z = o - m
        o_ref[...] = z - jnp.log(jnp.sum(jnp.exp(z), axis=1, keepdims=True))

    return pl.pallas_call(
        body,
        grid=(_N // _BN,),
        in_specs=[
            pl.BlockSpec((_NCORES, _BN, _RA), lambda i: (0, i, 0)),
            pl.BlockSpec((_BN, _RS), lambda i: (i, 0)),
            pl.BlockSpec((_BN, _RD), lambda i: (i, 0)),
            pl.BlockSpec((1, 64), lambda i: (0, 0)),
        ],
        out_specs=pl.BlockSpec((_BN, 64), lambda i: (i, 0)),
        out_shape=jax.ShapeDtypeStruct((_N, 64), jnp.float32),
    )(acc2, bysrc2, bydst2, b2)


def _head_mat(a):
    hh, cc = a.shape
    t = a[:, :, None] * jnp.eye(hh, dtype=a.dtype)[:, None, :]
    return t.reshape(hh * cc, hh)


_edge_pass_cache = {}


def _edge_pass(per_head):
    if per_head not in _edge_pass_cache:
        _edge_pass_cache[per_head] = _make_edge_pass(per_head)
    return _edge_pass_cache[per_head]


def kernel(x, edge_index, W1, a_src1, a_dst1, b1, W2, a_src2, a_dst2, b2):
    n_tiles = _NCORES * _NSUB
    n_chunks = _E // n_tiles // _CHUNK
    src = edge_index[0].reshape(n_tiles, n_chunks, _CHUNK)
    dst = edge_index[1].reshape(n_tiles, n_chunks, _CHUNK)

    f32 = W1.dtype
    zpad = jnp.zeros((64, 8), f32)
    A1 = W1 @ jnp.concatenate([jnp.eye(64, dtype=f32), _head_mat(a_src1),
                               zpad], axis=1)
    B1 = W1 @ jnp.concatenate([_head_mat(a_dst1), zpad], axis=1)
    E16 = jnp.concatenate(
        [jnp.repeat(jnp.eye(8, dtype=f32), 8, axis=1), jnp.zeros((8, 64), f32)],
        axis=0)
    s2 = W2 @ a_src2.T
    d2 = W2 @ a_dst2.T
    A2 = jnp.concatenate([W2, jnp.broadcast_to(s2, (64, 16))], axis=1)
    B2 = jnp.broadcast_to(d2, (64, 16))

    lanes = jnp.asarray(
        [[64 + 2 * j] * 8 + [64 + 2 * j + 1] * 8 for j in range(4)],
        dtype=jnp.int32)

    bysrc1, bydst1 = _prep1(x, A1, B1)
    acc1 = _edge_pass(True)(src, dst, bysrc1, bydst1, lanes)
    bysrc2, bydst2 = _finalize1_prep2(acc1, bysrc1, bydst1,
                                      b1.reshape(1, 64), E16, A2, B2)
    acc2 = _edge_pass(False)(src, dst, bysrc2, bydst2, lanes)
    return _finalize2(acc2, bysrc2, bydst2, b2.reshape(1, 64))


# speedup vs baseline: 176.2348x; 2.9173x over previous
import functools

import jax
import jax.numpy as jnp
from jax import lax
from jax.experimental import pallas as pl
from jax.experimental.pallas import tpu as pltpu
from jax.experimental.pallas import tpu_sc as plsc

_N = 10000
_E = 320000
_D = 128

_NCORES = 2
_NSUB = 16
_CHUNK = 80
_BN = 1000

_RS = 96
_RD = 16
_RA = 80


def _make_edge_pass(per_head):
    e_per_sc = _E // _NCORES
    e_per_tile = e_per_sc // _NSUB
    n_chunks = e_per_tile // _CHUNK
    rows_u = 624
    tail0 = rows_u * _NSUB
    tail_rows = _N - tail0
    zrows = 48

    mesh = plsc.VectorSubcoreMesh(core_axis_name="c", subcore_axis_name="s",
                                  num_cores=_NCORES, num_subcores=_NSUB)

    @functools.partial(
        pl.kernel,
        out_type=jax.ShapeDtypeStruct((_NCORES, _N, _RA), jnp.float32),
        mesh=mesh,
        compiler_params=pltpu.CompilerParams(use_tc_tiling_on_sc=False,
                                             needs_layout_passes=False),
        scratch_types=[
            pltpu.VMEM((n_chunks, _CHUNK), jnp.int32),
            pltpu.VMEM((n_chunks, _CHUNK), jnp.int32),
            pltpu.VMEM((_CHUNK, _RS), jnp.bfloat16),
            pltpu.VMEM((_CHUNK, _RS), jnp.bfloat16),
            pltpu.VMEM((_CHUNK, _RD), jnp.float32),
            pltpu.VMEM((_CHUNK, _RD), jnp.float32),
            pltpu.VMEM((_CHUNK, _RA), jnp.float32),
            pltpu.VMEM((_CHUNK, _RA), jnp.float32),
            pltpu.VMEM((zrows, _RA), jnp.float32),
            pltpu.VMEM((4, 16), jnp.int32),
            pltpu.VMEM_SHARED((_N, _RA), jnp.float32),
            pltpu.SemaphoreType.DMA,
            pltpu.SemaphoreType.DMA,
            pltpu.SemaphoreType.DMA,
            pltpu.SemaphoreType.DMA,
        ],
    )
    def edge_kernel(src_h, dst_h, bysrc_h, bydst_h, lanes_h, out_h,
                    src_i, dst_i, rsrc0, rsrc1, rdst0, rdst1, msg0, msg1,
                    zbuf, lanes_v, acc, gsem0, gsem1, ssem0, ssem1):
        cid = lax.axis_index("c")
        sid = lax.axis_index("s")
        tid = cid * _NSUB + sid
        row0 = sid * rows_u
        rsrc = (rsrc0, rsrc1)
        rdst = (rdst0, rdst1)
        msg = (msg0, msg1)
        gsem = (gsem0, gsem1)
        ssem = (ssem0, ssem1)

        idx_copies = [
            pltpu.async_copy(src_h.at[tid], src_i, gsem0),
            pltpu.async_copy(dst_h.at[tid], dst_i, gsem1),
        ]
        pltpu.sync_copy(lanes_h, lanes_v)

        zero16 = jnp.zeros((16,), jnp.float32)

        def zero_row(i, _):
            for j in range(_RA // 16):
                zbuf[i, pl.ds(16 * j, 16)] = zero16
            return 0

        lax.fori_loop(0, zrows, zero_row, 0)
        for k in range(rows_u // zrows):
            pltpu.sync_copy(zbuf, acc.at[pl.ds(row0 + k * zrows, zrows)])

        @pl.when(sid == _NSUB - 1)
        def _():
            pltpu.sync_copy(zbuf.at[pl.ds(0, tail_rows)],
                            acc.at[pl.ds(tail0, tail_rows)])

        for c in idx_copies:
            c.wait()
        plsc.subcore_barrier()

        def issue(c, b):
            pltpu.async_copy(bysrc_h.at[src_i.at[c]], rsrc[b], gsem[b])
            pltpu.async_copy(bydst_h.at[dst_i.at[c]], rdst[b], gsem[b])

        def wait_gathers(b):
            pltpu.make_async_copy(bysrc_h.at[src_i.at[0]], rsrc[b],
                                  gsem[b]).wait()
            pltpu.make_async_copy(bydst_h.at[dst_i.at[0]], rdst[b],
                                  gsem[b]).wait()

        def wait_scatter(b):
            pltpu.make_async_copy(msg[b], acc.at[dst_i.at[0]],
                                  ssem[b]).wait()

        unroll = 8

        def edge_body(b):
            lanes = [lanes_v[j, :] for j in range(4)]

            def one_edge(ei):
                s16, _ = plsc.unpack(rsrc[b][ei, pl.ds(64, 32)],
                                     format=plsc.PackFormat.INTERLEAVED,
                                     preferred_element_type=jnp.float32)
                a = s16 + rdst[b][ei, pl.ds(0, 16)]
                w = jnp.exp(jnp.maximum(a, 0.2 * a))
                msg[b][ei, pl.ds(64, 16)] = w
                for g in range(2):
                    h0, h1 = plsc.unpack(rsrc[b][ei, pl.ds(32 * g, 32)],
                                         format=plsc.PackFormat.INTERLEAVED,
                                         preferred_element_type=jnp.float32)
                    for jj, hv in ((2 * g, h0), (2 * g + 1, h1)):
                        if per_head:
                            wj = w.at[lanes[jj]].get(
                                mode="promise_in_bounds")
                        else:
                            wj = w
                        msg[b][ei, pl.ds(16 * jj, 16)] = hv * wj

            plsc.parallel_loop(0, _CHUNK, 1, unroll=unroll)(one_edge)

        issue(0, 0)
        issue(1, 1)
        wait_gathers(0)
        edge_body(0)
        pltpu.async_copy(msg0, acc.at[dst_i.at[0]], ssem0, add=True)

        def pair_body(k, _):
            c = 2 * k + 1

            @pl.when(c + 1 < n_chunks)
            def _():
                issue(c + 1, 0)
            wait_gathers(1)

            @pl.when(k > 0)
            def _():
                wait_scatter(1)
            edge_body(1)
            pltpu.async_copy(msg1, acc.at[dst_i.at[c]], ssem1, add=True)

            @pl.when(c + 2 < n_chunks)
            def _():
                issue(c + 2, 1)

            @pl.when(c + 1 < n_chunks)
            def _():
                wait_gathers(0)
                wait_scatter(0)
                edge_body(0)
                pltpu.async_copy(msg0, acc.at[dst_i.at[c + 1]], ssem0,
                                 add=True)
            return 0

        lax.fori_loop(0, (n_chunks - 1) // 2, pair_body, 0)
        wait_scatter(0)
        wait_scatter(1)
        plsc.subcore_barrier()
        pltpu.sync_copy(acc.at[pl.ds(row0, rows_u)],
                        out_h.at[cid, pl.ds(row0, rows_u)])

        @pl.when(sid == _NSUB - 1)
        def _():
            pltpu.sync_copy(acc.at[pl.ds(tail0, tail_rows)],
                            out_h.at[cid, pl.ds(tail0, tail_rows)])

    return edge_kernel


def _prep1(x, A1, B1):
    def body(x_ref, a_ref, b_ref, o1_ref, o2_ref):
        xv = x_ref[...]
        o1_ref[...] = jnp.dot(
            xv, a_ref[...],
            preferred_element_type=jnp.float32).astype(jnp.bfloat16)
        o2_ref[...] = jnp.dot(xv, b_ref[...], preferred_element_type=jnp.float32)

    return pl.pallas_call(
        body,
        grid=(_N // _BN,),
        in_specs=[
            pl.BlockSpec((_BN, _D), lambda i: (i, 0)),
            pl.BlockSpec((_D, _RS), lambda i: (0, 0)),
            pl.BlockSpec((_D, _RD), lambda i: (0, 0)),
        ],
        out_specs=[
            pl.BlockSpec((_BN, _RS), lambda i: (i, 0)),
            pl.BlockSpec((_BN, _RD), lambda i: (i, 0)),
        ],
        out_shape=[
            jax.ShapeDtypeStruct((_N, _RS), jnp.bfloat16),
            jax.ShapeDtypeStruct((_N, _RD), jnp.float32),
        ],
    )(x, A1, B1)


def _finalize1_prep2(acc1, bysrc1, bydst1, b1, E16, Pinv, A2, B2):
    def body(acc_ref, bs_ref, bd_ref, b1_ref, e_ref, p_ref, a2_ref, b2_ref,
             o1_ref, o2_ref):
        num = acc_ref[0, :, 0:64] + acc_ref[1, :, 0:64]
        den16 = acc_ref[0, :, 64:80] + acc_ref[1, :, 64:80]
        ev = e_ref[...]
        den = jnp.dot(den16, ev, preferred_element_type=jnp.float32)
        bs = jnp.dot(bs_ref[...].astype(jnp.float32), p_ref[...],
                     preferred_element_type=jnp.float32)
        h = bs[:, 0:64]
        a0 = bs[:, 64:80] + bd_ref[...]
        ws16 = jnp.exp(jnp.maximum(a0, 0.2 * a0))
        ws = jnp.dot(ws16, ev, preferred_element_type=jnp.float32)
        o = (num + ws * h) / (den + ws + 1e-16) + b1_ref[...]
        h1 = jnp.where(o > 0, o, jnp.exp(o) - 1.0)
        o1_ref[...] = jnp.dot(
            h1, a2_ref[...],
            preferred_element_type=jnp.float32).astype(jnp.bfloat16)
        o2_ref[...] = jnp.dot(h1, b2_ref[...], preferred_element_type=jnp.float32)

    return pl.pallas_call(
        body,
        grid=(_N // _BN,),
        in_specs=[
            pl.BlockSpec((_NCORES, _BN, _RA), lambda i: (0, i, 0)),
            pl.BlockSpec((_BN, _RS), lambda i: (i, 0)),
            pl.BlockSpec((_BN, _RD), lambda i: (i, 0)),
            pl.BlockSpec((1, 64), lambda i: (0, 0)),
            pl.BlockSpec((16, 64), lambda i: (0, 0)),
            pl.BlockSpec((_RS, _RS), lambda i: (0, 0)),
            pl.BlockSpec((64, _RS), lambda i: (0, 0)),
            pl.BlockSpec((64, _RD), lambda i: (0, 0)),
        ],
        out_specs=[
            pl.BlockSpec((_BN, _RS), lambda i: (i, 0)),
            pl.BlockSpec((_BN, _RD), lambda i: (i, 0)),
        ],
        out_shape=[
            jax.ShapeDtypeStruct((_N, _RS), jnp.bfloat16),
            jax.ShapeDtypeStruct((_N, _RD), jnp.float32),
        ],
    )(acc1, bysrc1, bydst1, b1, E16, Pinv, A2, B2)


def _finalize2(acc2, bysrc2, bydst2, b2, Pinv):
    def body(acc_ref, bs_ref, bd_ref, b2_ref, p_ref, o_ref):
        num = acc_ref[0, :, 0:64] + acc_ref[1, :, 0:64]
        den = acc_ref[0, :, 64:65] + acc_ref[1, :, 64:65]
        bs = jnp.dot(bs_ref[...].astype(jnp.float32), p_ref[...],
                     preferred_element_type=jnp.float32)
        h = bs[:, 0:64]
        a0 = bs[:, 64:65] + bd_ref[:, 0:1]
        ws = jnp.exp(jnp.maximum(a0, 0.2 * a0))
        o = (num + ws * h) / (den + ws + 1e-16) + b2_ref[...]
        m = jnp.max(o, axis=1, keepdims=True)
        z = o - m
        o_ref[...] = z - jnp.log(jnp.sum(jnp.exp(z), axis=1, keepdims=True))

    return pl.pallas_call(
        body,
        grid=(_N // _BN,),
        in_specs=[
            pl.BlockSpec((_NCORES, _BN, _RA), lambda i: (0, i, 0)),
            pl.BlockSpec((_BN, _RS), lambda i: (i, 0)),
            pl.BlockSpec((_BN, _RD), lambda i: (i, 0)),
            pl.BlockSpec((1, 64), lambda i: (0, 0)),
            pl.BlockSpec((_RS, _RS), lambda i: (0, 0)),
        ],
        out_specs=pl.BlockSpec((_BN, 64), lambda i: (i, 0)),
        out_shape=jax.ShapeDtypeStruct((_N, 64), jnp.float32),
    )(acc2, bysrc2, bydst2, b2, Pinv)


def _head_mat(a):
    hh, cc = a.shape
    t = a[:, :, None] * jnp.eye(hh, dtype=a.dtype)[:, None, :]
    return t.reshape(hh * cc, hh)


_edge_pass_cache = {}


def _edge_pass(per_head):
    if per_head not in _edge_pass_cache:
        _edge_pass_cache[per_head] = _make_edge_pass(per_head)
    return _edge_pass_cache[per_head]


def kernel(x, edge_index, W1, a_src1, a_dst1, b1, W2, a_src2, a_dst2, b2):
    n_tiles = _NCORES * _NSUB
    n_chunks = _E // n_tiles // _CHUNK
    src = edge_index[0].reshape(n_tiles, n_chunks, _CHUNK)
    dst = edge_index[1].reshape(n_tiles, n_chunks, _CHUNK)

    f32 = W1.dtype
    zpad = jnp.zeros((64, 8), f32)
    lperm = []
    for g in range(3):
        for k in range(16):
            lperm.extend([32 * g + k, 32 * g + 16 + k])
    lperm = jnp.asarray(lperm, dtype=jnp.int32)
    Pinv = jnp.zeros((_RS, _RS), f32).at[jnp.arange(_RS), lperm].set(1.0)

    A1log = W1 @ jnp.concatenate([jnp.eye(64, dtype=f32), _head_mat(a_src1),
                                  jnp.zeros((64, 24), f32)], axis=1)
    A1 = A1log[:, lperm]
    B1 = W1 @ jnp.concatenate([_head_mat(a_dst1), zpad], axis=1)
    E16 = jnp.concatenate(
        [jnp.repeat(jnp.eye(8, dtype=f32), 8, axis=1), jnp.zeros((8, 64), f32)],
        axis=0)
    s2 = W2 @ a_src2.T
    d2 = W2 @ a_dst2.T
    A2log = jnp.concatenate([W2, jnp.broadcast_to(s2, (64, 16)),
                             jnp.zeros((64, 16), f32)], axis=1)
    A2 = A2log[:, lperm]
    B2 = jnp.broadcast_to(d2, (64, 16))

    lanes = jnp.asarray(
        [[2 * j] * 8 + [2 * j + 1] * 8 for j in range(4)], dtype=jnp.int32)

    bysrc1, bydst1 = _prep1(x, A1, B1)
    acc1 = _edge_pass(True)(src, dst, bysrc1, bydst1, lanes)
    bysrc2, bydst2 = _finalize1_prep2(acc1, bysrc1, bydst1,
                                      b1.reshape(1, 64), E16, Pinv, A2, B2)
    acc2 = _edge_pass(False)(src, dst, bysrc2, bydst2, lanes)
    return _finalize2(acc2, bysrc2, bydst2, b2.reshape(1, 64), Pinv)


# speedup vs baseline: 198.7238x; 1.1276x over previous
import functools

import jax
import jax.numpy as jnp
from jax import lax
from jax.experimental import pallas as pl
from jax.experimental.pallas import tpu as pltpu
from jax.experimental.pallas import tpu_sc as plsc

_N = 10000
_E = 320000
_D = 128

_NCORES = 2
_NSUB = 16
_CHUNK = 80
_BN = 1000

_RS = 96
_RD = 16
_RA = 80


def _make_edge_pass(per_head):
    e_per_sc = _E // _NCORES
    e_per_tile = e_per_sc // _NSUB
    n_chunks = e_per_tile // _CHUNK
    rows_u = 624
    tail0 = rows_u * _NSUB
    tail_rows = _N - tail0
    zrows = 48

    mesh = plsc.VectorSubcoreMesh(core_axis_name="c", subcore_axis_name="s",
                                  num_cores=_NCORES, num_subcores=_NSUB)

    @functools.partial(
        pl.kernel,
        out_type=jax.ShapeDtypeStruct((_NCORES, _N, _RA), jnp.float32),
        mesh=mesh,
        compiler_params=pltpu.CompilerParams(use_tc_tiling_on_sc=False,
                                             needs_layout_passes=False),
        scratch_types=[
            pltpu.VMEM((n_chunks, _CHUNK), jnp.int32),
            pltpu.VMEM((n_chunks, _CHUNK), jnp.int32),
            pltpu.VMEM((_CHUNK, _RS), jnp.bfloat16),
            pltpu.VMEM((_CHUNK, _RS), jnp.bfloat16),
            pltpu.VMEM((_CHUNK, _RS), jnp.bfloat16),
            pltpu.VMEM((_CHUNK, _RS), jnp.bfloat16),
            pltpu.VMEM((_CHUNK, _RD), jnp.float32),
            pltpu.VMEM((_CHUNK, _RD), jnp.float32),
            pltpu.VMEM((_CHUNK, _RD), jnp.float32),
            pltpu.VMEM((_CHUNK, _RD), jnp.float32),
            pltpu.VMEM((_CHUNK, _RA), jnp.float32),
            pltpu.VMEM((_CHUNK, _RA), jnp.float32),
            pltpu.VMEM((_CHUNK, _RA), jnp.float32),
            pltpu.VMEM((_CHUNK, _RA), jnp.float32),
            pltpu.VMEM((zrows, _RA), jnp.float32),
            pltpu.VMEM((4, 16), jnp.int32),
            pltpu.VMEM_SHARED((_N, _RA), jnp.float32),
            pltpu.SemaphoreType.DMA,
            pltpu.SemaphoreType.DMA,
            pltpu.SemaphoreType.DMA,
            pltpu.SemaphoreType.DMA,
            pltpu.SemaphoreType.DMA,
            pltpu.SemaphoreType.DMA,
            pltpu.SemaphoreType.DMA,
            pltpu.SemaphoreType.DMA,
        ],
    )
    def edge_kernel(src_h, dst_h, bysrc_h, bydst_h, lanes_h, out_h,
                    src_i, dst_i, rsrc0, rsrc1, rsrc2, rsrc3,
                    rdst0, rdst1, rdst2, rdst3, msg0, msg1, msg2, msg3,
                    zbuf, lanes_v, acc, gsem0, gsem1, gsem2, gsem3,
                    ssem0, ssem1, ssem2, ssem3):
        cid = lax.axis_index("c")
        sid = lax.axis_index("s")
        tid = cid * _NSUB + sid
        row0 = sid * rows_u
        rsrc = (rsrc0, rsrc1, rsrc2, rsrc3)
        rdst = (rdst0, rdst1, rdst2, rdst3)
        msg = (msg0, msg1, msg2, msg3)
        gsem = (gsem0, gsem1, gsem2, gsem3)
        ssem = (ssem0, ssem1, ssem2, ssem3)

        idx_copies = [
            pltpu.async_copy(src_h.at[tid], src_i, gsem0),
            pltpu.async_copy(dst_h.at[tid], dst_i, gsem1),
        ]
        pltpu.sync_copy(lanes_h, lanes_v)

        zero16 = jnp.zeros((16,), jnp.float32)

        def zero_row(i, _):
            for j in range(_RA // 16):
                zbuf[i, pl.ds(16 * j, 16)] = zero16
            return 0

        lax.fori_loop(0, zrows, zero_row, 0)
        for k in range(rows_u // zrows):
            pltpu.sync_copy(zbuf, acc.at[pl.ds(row0 + k * zrows, zrows)])

        @pl.when(sid == _NSUB - 1)
        def _():
            pltpu.sync_copy(zbuf.at[pl.ds(0, tail_rows)],
                            acc.at[pl.ds(tail0, tail_rows)])

        for c in idx_copies:
            c.wait()
        plsc.subcore_barrier()

        def issue(c, b):
            pltpu.async_copy(bysrc_h.at[src_i.at[c]], rsrc[b], gsem[b])
            pltpu.async_copy(bydst_h.at[dst_i.at[c]], rdst[b], gsem[b])

        def wait_gathers(b):
            pltpu.make_async_copy(bysrc_h.at[src_i.at[0]], rsrc[b],
                                  gsem[b]).wait()
            pltpu.make_async_copy(bydst_h.at[dst_i.at[0]], rdst[b],
                                  gsem[b]).wait()

        def wait_scatter(b):
            pltpu.make_async_copy(msg[b], acc.at[dst_i.at[0]],
                                  ssem[b]).wait()

        unroll = 8

        def edge_body(b):
            lanes = [lanes_v[j, :] for j in range(4)]

            def one_edge(ei):
                s16, _ = plsc.unpack(rsrc[b][ei, pl.ds(64, 32)],
                                     format=plsc.PackFormat.INTERLEAVED,
                                     preferred_element_type=jnp.float32)
                a = s16 + rdst[b][ei, pl.ds(0, 16)]
                w = jnp.exp(jnp.maximum(a, 0.2 * a))
                msg[b][ei, pl.ds(64, 16)] = w
                for g in range(2):
                    h0, h1 = plsc.unpack(rsrc[b][ei, pl.ds(32 * g, 32)],
                                         format=plsc.PackFormat.INTERLEAVED,
                                         preferred_element_type=jnp.float32)
                    for jj, hv in ((2 * g, h0), (2 * g + 1, h1)):
                        if per_head:
                            wj = w.at[lanes[jj]].get(
                                mode="promise_in_bounds")
                        else:
                            wj = w
                        msg[b][ei, pl.ds(16 * jj, 16)] = hv * wj

            plsc.parallel_loop(0, _CHUNK, 1, unroll=unroll)(one_edge)

        def stage(c, b):
            wait_gathers(b)

            @pl.when(c >= 4)
            def _():
                wait_scatter(b)
            edge_body(b)
            pltpu.async_copy(msg[b], acc.at[dst_i.at[c]], ssem[b], add=True)

            @pl.when(c + 4 < n_chunks)
            def _():
                issue(c + 4, b)

        for b in range(4):
            issue(b, b)

        def quad_body(k, _):
            for u in range(4):
                stage(4 * k + u, u)
            return 0

        lax.fori_loop(0, (n_chunks - 1) // 4, quad_body, 0)
        stage(n_chunks - 1, 0)
        wait_scatter(1)
        wait_scatter(2)
        wait_scatter(3)
        wait_scatter(0)
        plsc.subcore_barrier()
        pltpu.sync_copy(acc.at[pl.ds(row0, rows_u)],
                        out_h.at[cid, pl.ds(row0, rows_u)])

        @pl.when(sid == _NSUB - 1)
        def _():
            pltpu.sync_copy(acc.at[pl.ds(tail0, tail_rows)],
                            out_h.at[cid, pl.ds(tail0, tail_rows)])

    return edge_kernel


def _prep1(x, A1, B1):
    def body(x_ref, a_ref, b_ref, o1_ref, o2_ref):
        xv = x_ref[...]
        o1_ref[...] = jnp.dot(
            xv, a_ref[...],
            preferred_element_type=jnp.float32).astype(jnp.bfloat16)
        o2_ref[...] = jnp.dot(xv, b_ref[...], preferred_element_type=jnp.float32)

    return pl.pallas_call(
        body,
        grid=(_N // _BN,),
        in_specs=[
            pl.BlockSpec((_BN, _D), lambda i: (i, 0)),
            pl.BlockSpec((_D, _RS), lambda i: (0, 0)),
            pl.BlockSpec((_D, _RD), lambda i: (0, 0)),
        ],
        out_specs=[
            pl.BlockSpec((_BN, _RS), lambda i: (i, 0)),
            pl.BlockSpec((_BN, _RD), lambda i: (i, 0)),
        ],
        out_shape=[
            jax.ShapeDtypeStruct((_N, _RS), jnp.bfloat16),
            jax.ShapeDtypeStruct((_N, _RD), jnp.float32),
        ],
    )(x, A1, B1)


def _finalize1_prep2(acc1, bysrc1, bydst1, b1, E16, Pinv, A2, B2):
    def body(acc_ref, bs_ref, bd_ref, b1_ref, e_ref, p_ref, a2_ref, b2_ref,
             o1_ref, o2_ref):
        num = acc_ref[0, :, 0:64] + acc_ref[1, :, 0:64]
        den16 = acc_ref[0, :, 64:80] + acc_ref[1, :, 64:80]
        ev = e_ref[...]
        den = jnp.dot(den16, ev, preferred_element_type=jnp.float32)
        bs = jnp.dot(bs_ref[...].astype(jnp.float32), p_ref[...],
                     preferred_element_type=jnp.float32)
        h = bs[:, 0:64]
        a0 = bs[:, 64:80] + bd_ref[...]
        ws16 = jnp.exp(jnp.maximum(a0, 0.2 * a0))
        ws = jnp.dot(ws16, ev, preferred_element_type=jnp.float32)
        o = (num + ws * h) / (den + ws + 1e-16) + b1_ref[...]
        h1 = jnp.where(o > 0, o, jnp.exp(o) - 1.0)
        o1_ref[...] = jnp.dot(
            h1, a2_ref[...],
            preferred_element_type=jnp.float32).astype(jnp.bfloat16)
        o2_ref[...] = jnp.dot(h1, b2_ref[...], preferred_element_type=jnp.float32)

    return pl.pallas_call(
        body,
        grid=(_N // _BN,),
        in_specs=[
            pl.BlockSpec((_NCORES, _BN, _RA), lambda i: (0, i, 0)),
            pl.BlockSpec((_BN, _RS), lambda i: (i, 0)),
            pl.BlockSpec((_BN, _RD), lambda i: (i, 0)),
            pl.BlockSpec((1, 64), lambda i: (0, 0)),
            pl.BlockSpec((16, 64), lambda i: (0, 0)),
            pl.BlockSpec((_RS, _RS), lambda i: (0, 0)),
            pl.BlockSpec((64, _RS), lambda i: (0, 0)),
            pl.BlockSpec((64, _RD), lambda i: (0, 0)),
        ],
        out_specs=[
            pl.BlockSpec((_BN, _RS), lambda i: (i, 0)),
            pl.BlockSpec((_BN, _RD), lambda i: (i, 0)),
        ],
        out_shape=[
            jax.ShapeDtypeStruct((_N, _RS), jnp.bfloat16),
            jax.ShapeDtypeStruct((_N, _RD), jnp.float32),
        ],
    )(acc1, bysrc1, bydst1, b1, E16, Pinv, A2, B2)


def _finalize2(acc2, bysrc2, bydst2, b2, Pinv):
    def body(acc_ref, bs_ref, bd_ref, b2_ref, p_ref, o_ref):
        num = acc_ref[0, :, 0:64] + acc_ref[1, :, 0:64]
        den = acc_ref[0, :, 64:65] + acc_ref[1, :, 64:65]
        bs = jnp.dot(bs_ref[...].astype(jnp.float32), p_ref[...],
                     preferred_element_type=jnp.float32)
        h = bs[:, 0:64]
        a0 = bs[:, 64:65] + bd_ref[:, 0:1]
        ws = jnp.exp(jnp.maximum(a0, 0.2 * a0))
        o = (num + ws * h) / (den + ws + 1e-16) + b2_ref[...]
        m = jnp.max(o, axis=1, keepdims=True)
        z = o - m
        o_ref[...] = z - jnp.log(jnp.sum(jnp.exp(z), axis=1, keepdims=True))

    return pl.pallas_call(
        body,
        grid=(_N // _BN,),
        in_specs=[
            pl.BlockSpec((_NCORES, _BN, _RA), lambda i: (0, i, 0)),
            pl.BlockSpec((_BN, _RS), lambda i: (i, 0)),
            pl.BlockSpec((_BN, _RD), lambda i: (i, 0)),
            pl.BlockSpec((1, 64), lambda i: (0, 0)),
            pl.BlockSpec((_RS, _RS), lambda i: (0, 0)),
        ],
        out_specs=pl.BlockSpec((_BN, 64), lambda i: (i, 0)),
        out_shape=jax.ShapeDtypeStruct((_N, 64), jnp.float32),
    )(acc2, bysrc2, bydst2, b2, Pinv)


def _head_mat(a):
    hh, cc = a.shape
    t = a[:, :, None] * jnp.eye(hh, dtype=a.dtype)[:, None, :]
    return t.reshape(hh * cc, hh)


_edge_pass_cache = {}


def _edge_pass(per_head):
    if per_head not in _edge_pass_cache:
        _edge_pass_cache[per_head] = _make_edge_pass(per_head)
    return _edge_pass_cache[per_head]


def kernel(x, edge_index, W1, a_src1, a_dst1, b1, W2, a_src2, a_dst2, b2):
    n_tiles = _NCORES * _NSUB
    n_chunks = _E // n_tiles // _CHUNK
    src = edge_index[0].reshape(n_tiles, n_chunks, _CHUNK)
    dst = edge_index[1].reshape(n_tiles, n_chunks, _CHUNK)

    f32 = W1.dtype
    zpad = jnp.zeros((64, 8), f32)
    lperm = []
    for g in range(3):
        for k in range(16):
            lperm.extend([32 * g + k, 32 * g + 16 + k])
    lperm = jnp.asarray(lperm, dtype=jnp.int32)
    Pinv = jnp.zeros((_RS, _RS), f32).at[jnp.arange(_RS), lperm].set(1.0)

    A1log = W1 @ jnp.concatenate([jnp.eye(64, dtype=f32), _head_mat(a_src1),
                                  jnp.zeros((64, 24), f32)], axis=1)
    A1 = A1log[:, lperm]
    B1 = W1 @ jnp.concatenate([_head_mat(a_dst1), zpad], axis=1)
    E16 = jnp.concatenate(
        [jnp.repeat(jnp.eye(8, dtype=f32), 8, axis=1), jnp.zeros((8, 64), f32)],
        axis=0)
    s2 = W2 @ a_src2.T
    d2 = W2 @ a_dst2.T
    A2log = jnp.concatenate([W2, jnp.broadcast_to(s2, (64, 16)),
                             jnp.zeros((64, 16), f32)], axis=1)
    A2 = A2log[:, lperm]
    B2 = jnp.broadcast_to(d2, (64, 16))

    lanes = jnp.asarray(
        [[2 * j] * 8 + [2 * j + 1] * 8 for j in range(4)], dtype=jnp.int32)

    bysrc1, bydst1 = _prep1(x, A1, B1)
    acc1 = _edge_pass(True)(src, dst, bysrc1, bydst1, lanes)
    bysrc2, bydst2 = _finalize1_prep2(acc1, bysrc1, bydst1,
                                      b1.reshape(1, 64), E16, Pinv, A2, B2)
    acc2 = _edge_pass(False)(src, dst, bysrc2, bydst2, lanes)
    return _finalize2(acc2, bysrc2, bydst2, b2.reshape(1, 64), Pinv)


# speedup vs baseline: 202.0598x; 1.0168x over previous
import functools

import jax
import jax.numpy as jnp
from jax import lax
from jax.experimental import pallas as pl
from jax.experimental.pallas import tpu as pltpu
from jax.experimental.pallas import tpu_sc as plsc

_N = 10000
_E = 320000
_D = 128

_NCORES = 2
_NSUB = 16
_CHUNK = 80
_BN = 1000

_RS = 96
_RD = 16
_RA = 80


def _make_edge_pass(per_head):
    e_per_sc = _E // _NCORES
    e_per_tile = e_per_sc // _NSUB
    n_chunks = e_per_tile // _CHUNK
    rows_u = 624
    tail0 = rows_u * _NSUB
    tail_rows = _N - tail0
    zrows = 48

    mesh = plsc.VectorSubcoreMesh(core_axis_name="c", subcore_axis_name="s",
                                  num_cores=_NCORES, num_subcores=_NSUB)

    @functools.partial(
        pl.kernel,
        out_type=jax.ShapeDtypeStruct((_NCORES, _N, _RA), jnp.float32),
        mesh=mesh,
        compiler_params=pltpu.CompilerParams(use_tc_tiling_on_sc=False,
                                             needs_layout_passes=False),
        scratch_types=[
            pltpu.VMEM((n_chunks, _CHUNK), jnp.int32),
            pltpu.VMEM((n_chunks, _CHUNK), jnp.int32),
            pltpu.VMEM((_CHUNK, _RS), jnp.bfloat16),
            pltpu.VMEM((_CHUNK, _RS), jnp.bfloat16),
            pltpu.VMEM((_CHUNK, _RS), jnp.bfloat16),
            pltpu.VMEM((_CHUNK, _RS), jnp.bfloat16),
            pltpu.VMEM((_CHUNK, _RD), jnp.float32),
            pltpu.VMEM((_CHUNK, _RD), jnp.float32),
            pltpu.VMEM((_CHUNK, _RD), jnp.float32),
            pltpu.VMEM((_CHUNK, _RD), jnp.float32),
            pltpu.VMEM((_CHUNK, _RA), jnp.float32),
            pltpu.VMEM((_CHUNK, _RA), jnp.float32),
            pltpu.VMEM((_CHUNK, _RA), jnp.float32),
            pltpu.VMEM((_CHUNK, _RA), jnp.float32),
            pltpu.VMEM((zrows, _RA), jnp.float32),
            pltpu.VMEM((4, 16), jnp.int32),
            pltpu.VMEM_SHARED((_N, _RA), jnp.float32),
            pltpu.SemaphoreType.DMA,
            pltpu.SemaphoreType.DMA,
            pltpu.SemaphoreType.DMA,
            pltpu.SemaphoreType.DMA,
            pltpu.SemaphoreType.DMA,
            pltpu.SemaphoreType.DMA,
            pltpu.SemaphoreType.DMA,
            pltpu.SemaphoreType.DMA,
        ],
    )
    def edge_kernel(src_h, dst_h, bysrc_h, bydst_h, lanes_h, out_h,
                    src_i, dst_i, rsrc0, rsrc1, rsrc2, rsrc3,
                    rdst0, rdst1, rdst2, rdst3, msg0, msg1, msg2, msg3,
                    zbuf, lanes_v, acc, gsem0, gsem1, gsem2, gsem3,
                    ssem0, ssem1, ssem2, ssem3):
        cid = lax.axis_index("c")
        sid = lax.axis_index("s")
        tid = cid * _NSUB + sid
        row0 = sid * rows_u
        rsrc = (rsrc0, rsrc1, rsrc2, rsrc3)
        rdst = (rdst0, rdst1, rdst2, rdst3)
        msg = (msg0, msg1, msg2, msg3)
        gsem = (gsem0, gsem1, gsem2, gsem3)
        ssem = (ssem0, ssem1, ssem2, ssem3)

        idx_copies = [
            pltpu.async_copy(src_h.at[tid], src_i, gsem0),
            pltpu.async_copy(dst_h.at[tid], dst_i, gsem1),
        ]
        pltpu.sync_copy(lanes_h, lanes_v)

        zero16 = jnp.zeros((16,), jnp.float32)

        def zero_row(i, _):
            for j in range(_RA // 16):
                zbuf[i, pl.ds(16 * j, 16)] = zero16
            return 0

        def issue(c, b):
            pltpu.async_copy(bysrc_h.at[src_i.at[c]], rsrc[b], gsem[b])
            pltpu.async_copy(bydst_h.at[dst_i.at[c]], rdst[b], gsem[b])

        def wait_gathers(b):
            pltpu.make_async_copy(bysrc_h.at[src_i.at[0]], rsrc[b],
                                  gsem[b]).wait()
            pltpu.make_async_copy(bydst_h.at[dst_i.at[0]], rdst[b],
                                  gsem[b]).wait()

        def wait_scatter(b):
            pltpu.make_async_copy(msg[b], acc.at[dst_i.at[0]],
                                  ssem[b]).wait()

        unroll = 8

        def edge_body(b):
            lanes = [lanes_v[j, :] for j in range(4)]

            def one_edge(ei):
                s16, _ = plsc.unpack(rsrc[b][ei, pl.ds(64, 32)],
                                     format=plsc.PackFormat.INTERLEAVED,
                                     preferred_element_type=jnp.float32)
                a = s16 + rdst[b][ei, pl.ds(0, 16)]
                w = jnp.exp(jnp.maximum(a, 0.2 * a))
                msg[b][ei, pl.ds(64, 16)] = w
                for g in range(2):
                    h0, h1 = plsc.unpack(rsrc[b][ei, pl.ds(32 * g, 32)],
                                         format=plsc.PackFormat.INTERLEAVED,
                                         preferred_element_type=jnp.float32)
                    for jj, hv in ((2 * g, h0), (2 * g + 1, h1)):
                        if per_head:
                            wj = w.at[lanes[jj]].get(
                                mode="promise_in_bounds")
                        else:
                            wj = w
                        msg[b][ei, pl.ds(16 * jj, 16)] = hv * wj

            plsc.parallel_loop(0, _CHUNK, 1, unroll=unroll)(one_edge)

        lax.fori_loop(0, zrows, zero_row, 0)
        for c in idx_copies:
            c.wait()
        for b in range(4):
            issue(b, b)
        for k in range(rows_u // zrows):
            pltpu.sync_copy(zbuf, acc.at[pl.ds(row0 + k * zrows, zrows)])

        @pl.when(sid == _NSUB - 1)
        def _():
            pltpu.sync_copy(zbuf.at[pl.ds(0, tail_rows)],
                            acc.at[pl.ds(tail0, tail_rows)])

        plsc.subcore_barrier()

        def stage(c, b):
            wait_gathers(b)

            @pl.when(c >= 4)
            def _():
                wait_scatter(b)
            edge_body(b)
            pltpu.async_copy(msg[b], acc.at[dst_i.at[c]], ssem[b], add=True)

            @pl.when(c + 4 < n_chunks)
            def _():
                issue(c + 4, b)

        def quad_body(k, _):
            for u in range(4):
                stage(4 * k + u, u)
            return 0

        lax.fori_loop(0, (n_chunks - 1) // 4, quad_body, 0)
        stage(n_chunks - 1, 0)
        wait_scatter(1)
        wait_scatter(2)
        wait_scatter(3)
        wait_scatter(0)
        plsc.subcore_barrier()
        pltpu.sync_copy(acc.at[pl.ds(row0, rows_u)],
                        out_h.at[cid, pl.ds(row0, rows_u)])

        @pl.when(sid == _NSUB - 1)
        def _():
            pltpu.sync_copy(acc.at[pl.ds(tail0, tail_rows)],
                            out_h.at[cid, pl.ds(tail0, tail_rows)])

    return edge_kernel


def _prep1(x, A1, B1):
    def body(x_ref, a_ref, b_ref, o1_ref, o2_ref):
        xv = x_ref[...]
        o1_ref[...] = jnp.dot(
            xv, a_ref[...],
            preferred_element_type=jnp.float32).astype(jnp.bfloat16)
        o2_ref[...] = jnp.dot(xv, b_ref[...], preferred_element_type=jnp.float32)

    return pl.pallas_call(
        body,
        grid=(_N // _BN,),
        in_specs=[
            pl.BlockSpec((_BN, _D), lambda i: (i, 0)),
            pl.BlockSpec((_D, _RS), lambda i: (0, 0)),
            pl.BlockSpec((_D, _RD), lambda i: (0, 0)),
        ],
        out_specs=[
            pl.BlockSpec((_BN, _RS), lambda i: (i, 0)),
            pl.BlockSpec((_BN, _RD), lambda i: (i, 0)),
        ],
        out_shape=[
            jax.ShapeDtypeStruct((_N, _RS), jnp.bfloat16),
            jax.ShapeDtypeStruct((_N, _RD), jnp.float32),
        ],
    )(x, A1, B1)


def _finalize1_prep2(acc1, bysrc1, bydst1, b1, E16, Pinv, A2, B2):
    def body(acc_ref, bs_ref, bd_ref, b1_ref, e_ref, p_ref, a2_ref, b2_ref,
             o1_ref, o2_ref):
        num = acc_ref[0, :, 0:64] + acc_ref[1, :, 0:64]
        den16 = acc_ref[0, :, 64:80] + acc_ref[1, :, 64:80]
        ev = e_ref[...]
        den = jnp.dot(den16, ev, preferred_element_type=jnp.float32)
        bs = jnp.dot(bs_ref[...].astype(jnp.float32), p_ref[...],
                     preferred_element_type=jnp.float32)
        h = bs[:, 0:64]
        a0 = bs[:, 64:80] + bd_ref[...]
        ws16 = jnp.exp(jnp.maximum(a0, 0.2 * a0))
        ws = jnp.dot(ws16, ev, preferred_element_type=jnp.float32)
        o = (num + ws * h) / (den + ws + 1e-16) + b1_ref[...]
        h1 = jnp.where(o > 0, o, jnp.exp(o) - 1.0)
        o1_ref[...] = jnp.dot(
            h1, a2_ref[...],
            preferred_element_type=jnp.float32).astype(jnp.bfloat16)
        o2_ref[...] = jnp.dot(h1, b2_ref[...], preferred_element_type=jnp.float32)

    return pl.pallas_call(
        body,
        grid=(_N // _BN,),
        in_specs=[
            pl.BlockSpec((_NCORES, _BN, _RA), lambda i: (0, i, 0)),
            pl.BlockSpec((_BN, _RS), lambda i: (i, 0)),
            pl.BlockSpec((_BN, _RD), lambda i: (i, 0)),
            pl.BlockSpec((1, 64), lambda i: (0, 0)),
            pl.BlockSpec((16, 64), lambda i: (0, 0)),
            pl.BlockSpec((_RS, _RS), lambda i: (0, 0)),
            pl.BlockSpec((64, _RS), lambda i: (0, 0)),
            pl.BlockSpec((64, _RD), lambda i: (0, 0)),
        ],
        out_specs=[
            pl.BlockSpec((_BN, _RS), lambda i: (i, 0)),
            pl.BlockSpec((_BN, _RD), lambda i: (i, 0)),
        ],
        out_shape=[
            jax.ShapeDtypeStruct((_N, _RS), jnp.bfloat16),
            jax.ShapeDtypeStruct((_N, _RD), jnp.float32),
        ],
    )(acc1, bysrc1, bydst1, b1, E16, Pinv, A2, B2)


def _finalize2(acc2, bysrc2, bydst2, b2, Pinv):
    def body(acc_ref, bs_ref, bd_ref, b2_ref, p_ref, o_ref):
        num = acc_ref[0, :, 0:64] + acc_ref[1, :, 0:64]
        den = acc_ref[0, :, 64:65] + acc_ref[1, :, 64:65]
        bs = jnp.dot(bs_ref[...].astype(jnp.float32), p_ref[...],
                     preferred_element_type=jnp.float32)
        h = bs[:, 0:64]
        a0 = bs[:, 64:65] + bd_ref[:, 0:1]
        ws = jnp.exp(jnp.maximum(a0, 0.2 * a0))
        o = (num + ws * h) / (den + ws + 1e-16) + b2_ref[...]
        m = jnp.max(o, axis=1, keepdims=True)
        z = o - m
        o_ref[...] = z - jnp.log(jnp.sum(jnp.exp(z), axis=1, keepdims=True))

    return pl.pallas_call(
        body,
        grid=(_N // _BN,),
        in_specs=[
            pl.BlockSpec((_NCORES, _BN, _RA), lambda i: (0, i, 0)),
            pl.BlockSpec((_BN, _RS), lambda i: (i, 0)),
            pl.BlockSpec((_BN, _RD), lambda i: (i, 0)),
            pl.BlockSpec((1, 64), lambda i: (0, 0)),
            pl.BlockSpec((_RS, _RS), lambda i: (0, 0)),
        ],
        out_specs=pl.BlockSpec((_BN, 64), lambda i: (i, 0)),
        out_shape=jax.ShapeDtypeStruct((_N, 64), jnp.float32),
    )(acc2, bysrc2, bydst2, b2, Pinv)


def _head_mat(a):
    hh, cc = a.shape
    t = a[:, :, None] * jnp.eye(hh, dtype=a.dtype)[:, None, :]
    return t.reshape(hh * cc, hh)


_edge_pass_cache = {}


def _edge_pass(per_head):
    if per_head not in _edge_pass_cache:
        _edge_pass_cache[per_head] = _make_edge_pass(per_head)
    return _edge_pass_cache[per_head]


def kernel(x, edge_index, W1, a_src1, a_dst1, b1, W2, a_src2, a_dst2, b2):
    n_tiles = _NCORES * _NSUB
    n_chunks = _E // n_tiles // _CHUNK
    src = edge_index[0].reshape(n_tiles, n_chunks, _CHUNK)
    dst = edge_index[1].reshape(n_tiles, n_chunks, _CHUNK)

    f32 = W1.dtype
    zpad = jnp.zeros((64, 8), f32)
    lperm = []
    for g in range(3):
        for k in range(16):
            lperm.extend([32 * g + k, 32 * g + 16 + k])
    lperm = jnp.asarray(lperm, dtype=jnp.int32)
    Pinv = jnp.zeros((_RS, _RS), f32).at[jnp.arange(_RS), lperm].set(1.0)

    A1log = W1 @ jnp.concatenate([jnp.eye(64, dtype=f32), _head_mat(a_src1),
                                  jnp.zeros((64, 24), f32)], axis=1)
    A1 = A1log[:, lperm]
    B1 = W1 @ jnp.concatenate([_head_mat(a_dst1), zpad], axis=1)
    E16 = jnp.concatenate(
        [jnp.repeat(jnp.eye(8, dtype=f32), 8, axis=1), jnp.zeros((8, 64), f32)],
        axis=0)
    s2 = W2 @ a_src2.T
    d2 = W2 @ a_dst2.T
    A2log = jnp.concatenate([W2, jnp.broadcast_to(s2, (64, 16)),
                             jnp.zeros((64, 16), f32)], axis=1)
    A2 = A2log[:, lperm]
    B2 = jnp.broadcast_to(d2, (64, 16))

    lanes = jnp.asarray(
        [[2 * j] * 8 + [2 * j + 1] * 8 for j in range(4)], dtype=jnp.int32)

    bysrc1, bydst1 = _prep1(x, A1, B1)
    acc1 = _edge_pass(True)(src, dst, bysrc1, bydst1, lanes)
    bysrc2, bydst2 = _finalize1_prep2(acc1, bysrc1, bydst1,
                                      b1.reshape(1, 64), E16, Pinv, A2, B2)
    acc2 = _edge_pass(False)(src, dst, bysrc2, bydst2, lanes)
    return _finalize2(acc2, bysrc2, bydst2, b2.reshape(1, 64), Pinv)
